# speedup vs baseline: 2.5313x; 2.5313x over previous
import functools
import math

import jax
import jax.numpy as jnp
from jax import lax
from jax.experimental import pallas as pl
from jax.experimental.pallas import tpu as pltpu
from jax.experimental.pallas import tpu_sc as plsc

D_MODEL = 128
_SCALE = math.sqrt(float(D_MODEL))

_NC = 2
_NS = 16
_NW = _NC * _NS

_B = 4096 * 50
_BPW = _B // _NW
_CHUNK = 320
_NCHUNK = _BPW // _CHUNK

_mesh = plsc.VectorSubcoreMesh(core_axis_name="c", subcore_axis_name="s")


@functools.partial(
    pl.kernel,
    mesh=_mesh,
    out_type=jax.ShapeDtypeStruct((_B, D_MODEL), jnp.float32),
    scratch_types=[
        pltpu.VMEM((_CHUNK,), jnp.int32),
        pltpu.VMEM((_CHUNK, D_MODEL), jnp.float32),
        pltpu.SemaphoreType.DMA,
    ],
)
def _gather_scale(idx_hbm, table_hbm, out_hbm, idx_v, rows_v, sem):
    wid = lax.axis_index("s") * _NC + lax.axis_index("c")
    base = wid * _BPW
    scale = jnp.float32(_SCALE)

    def chunk_body(j, carry):
        off = base + j * _CHUNK
        pltpu.sync_copy(idx_hbm.at[pl.ds(off, _CHUNK)], idx_v)
        pltpu.async_copy(table_hbm.at[idx_v], rows_v, sem).wait()

        def row_body(r, c2):
            for k in range(D_MODEL // 16):
                sl = pl.ds(k * 16, 16)
                rows_v[r, sl] = rows_v[r, sl] * scale
            return c2

        lax.fori_loop(0, _CHUNK, row_body, 0)
        pltpu.sync_copy(rows_v, out_hbm.at[pl.ds(off, _CHUNK)])
        return carry

    lax.fori_loop(0, _NCHUNK, chunk_body, 0)


def kernel(idxs, emb_table):
    flat = idxs.reshape(-1).astype(jnp.int32)
    out = _gather_scale(flat, emb_table)
    return out.reshape(idxs.shape + (D_MODEL,))


# speedup vs baseline: 2.9470x; 1.1642x over previous
import functools
import math

import jax
import jax.numpy as jnp
from jax import lax
from jax.experimental import pallas as pl
from jax.experimental.pallas import tpu as pltpu
from jax.experimental.pallas import tpu_sc as plsc

D_MODEL = 128
_SCALE = math.sqrt(float(D_MODEL))

_NC = 2
_NS = 16
_NW = _NC * _NS

_B = 4096 * 50
_BPW = _B // _NW
_CHUNK = 200
_NCHUNK = _BPW // _CHUNK
_NBUF = 4

_mesh = plsc.VectorSubcoreMesh(core_axis_name="c", subcore_axis_name="s")


@functools.partial(
    pl.kernel,
    mesh=_mesh,
    out_type=jax.ShapeDtypeStruct((_B, D_MODEL), jnp.float32),
    scratch_types=(
        [pltpu.VMEM((_BPW,), jnp.int32)]
        + [pltpu.VMEM((_CHUNK, D_MODEL), jnp.float32) for _ in range(_NBUF)]
        + [pltpu.SemaphoreType.DMA for _ in range(2 * _NBUF)]
    ),
)
def _gather_scale(idx_hbm, table_hbm, out_hbm, idx_v, r0, r1, r2, r3,
                  g0, g1, g2, g3, s0, s1, s2, s3):
    bufs = (r0, r1, r2, r3)
    gsems = (g0, g1, g2, g3)
    ssems = (s0, s1, s2, s3)
    wid = lax.axis_index("s") * _NC + lax.axis_index("c")
    base = wid * _BPW
    scale = jnp.float32(_SCALE)

    pltpu.sync_copy(idx_hbm.at[pl.ds(base, _BPW)], idx_v)

    def start_gather(cidx, b):
        pltpu.async_copy(
            table_hbm.at[idx_v.at[pl.ds(cidx * _CHUNK, _CHUNK)]],
            bufs[b], gsems[b])

    for c in range(_NBUF - 1):
        start_gather(c, c)

    @pl.loop(0, _NCHUNK, step=_NBUF)
    def _chunks(j):
        for b in range(_NBUF):
            cidx = j + b
            nb = (b + _NBUF - 1) % _NBUF
            @pl.when(cidx + _NBUF - 1 < _NCHUNK)
            def _():
                @pl.when(cidx >= 1)
                def _():
                    pltpu.make_async_copy(
                        bufs[nb],
                        out_hbm.at[pl.ds(base, _CHUNK)],
                        ssems[nb]).wait()
                start_gather(cidx + _NBUF - 1, nb)

            pltpu.make_async_copy(
                table_hbm.at[idx_v.at[pl.ds(0, _CHUNK)]],
                bufs[b], gsems[b]).wait()

            @plsc.parallel_loop(0, _CHUNK, unroll=2)
            def _rows(r):
                for k in range(D_MODEL // 16):
                    sl = pl.ds(k * 16, 16)
                    bufs[b][r, sl] = bufs[b][r, sl] * scale

            pltpu.async_copy(
                bufs[b],
                out_hbm.at[pl.ds(base + cidx * _CHUNK, _CHUNK)],
                ssems[b])

    for c in range(_NCHUNK - _NBUF, _NCHUNK):
        b = c % _NBUF
        pltpu.make_async_copy(
            bufs[b], out_hbm.at[pl.ds(base, _CHUNK)], ssems[b]).wait()


def kernel(idxs, emb_table):
    flat = idxs.reshape(-1).astype(jnp.int32)
    out = _gather_scale(flat, emb_table)
    return out.reshape(idxs.shape + (D_MODEL,))


# speedup vs baseline: 5.2233x; 1.7724x over previous
import math

import jax
import jax.numpy as jnp
from jax import lax
from jax.experimental import pallas as pl
from jax.experimental.pallas import tpu as pltpu
from jax.experimental.pallas import tpu_sc as plsc

D_MODEL = 128
_SCALE = math.sqrt(float(D_MODEL))

_NC = 2
_NS = 16
_NW = _NC * _NS

_NROWS = 4096
_K = 50
_RPW = _NROWS // _NW
_RC = 4
_NCHUNK = _RPW // _RC
_NBUF = 4

_mesh = plsc.VectorSubcoreMesh(core_axis_name="c", subcore_axis_name="s")


@pl.kernel(
    out_type=jax.ShapeDtypeStruct((_NROWS, _K, D_MODEL), jnp.float32),
    mesh=_mesh,
    scratch_types=(
        [pltpu.VMEM((_RPW, _K), jnp.int32)]
        + [pltpu.VMEM((_RC, _K, D_MODEL), jnp.float32) for _ in range(_NBUF)]
        + [pltpu.SemaphoreType.DMA for _ in range(2 * _NBUF)]
    ),
    compiler_params=pltpu.CompilerParams(use_tc_tiling_on_sc=True),
)
def _gather_scale(idx_hbm, table_hbm, out_hbm, idx_v, r0, r1, r2, r3,
                  g0, g1, g2, g3, s0, s1, s2, s3):
    bufs = (r0, r1, r2, r3)
    gsems = (g0, g1, g2, g3)
    ssems = (s0, s1, s2, s3)
    wid = lax.axis_index("s") * _NC + lax.axis_index("c")
    base_row = wid * _RPW
    scale = jnp.float32(_SCALE)

    pltpu.sync_copy(idx_hbm.at[pl.ds(base_row, _RPW)], idx_v)

    def start_gather(cidx, b):
        for rr in range(_RC):
            pltpu.async_copy(
                table_hbm.at[idx_v.at[cidx * _RC + rr]],
                bufs[b].at[rr], gsems[b])

    def wait_gather(b):
        for rr in range(_RC):
            pltpu.make_async_copy(
                table_hbm.at[idx_v.at[0]], bufs[b].at[rr], gsems[b]).wait()

    def wait_store(b):
        pltpu.make_async_copy(
            bufs[b], out_hbm.at[pl.ds(base_row, _RC)], ssems[b]).wait()

    for c in range(_NBUF - 1):
        start_gather(c, c)

    @pl.loop(0, _NCHUNK, step=_NBUF)
    def _chunks(j):
        for b in range(_NBUF):
            cidx = j + b
            nb = (b + _NBUF - 1) % _NBUF
            @pl.when(cidx + _NBUF - 1 < _NCHUNK)
            def _():
                @pl.when(cidx >= 1)
                def _():
                    wait_store(nb)
                start_gather(cidx + _NBUF - 1, nb)

            wait_gather(b)

            for rr in range(_RC):
                @plsc.parallel_loop(0, _K, unroll=2)
                def _rows(i):
                    for k in range(D_MODEL // 16):
                        sl = pl.ds(k * 16, 16)
                        bufs[b][rr, i, sl] = bufs[b][rr, i, sl] * scale

            pltpu.async_copy(
                bufs[b],
                out_hbm.at[pl.ds(base_row + cidx * _RC, _RC)],
                ssems[b])

    for c in range(_NCHUNK - _NBUF, _NCHUNK):
        wait_store(c % _NBUF)


def kernel(idxs, emb_table):
    return _gather_scale(idxs.astype(jnp.int32), emb_table)


# speedup vs baseline: 9.4275x; 1.8049x over previous
import math

import jax
import jax.numpy as jnp
from jax import lax
from jax.experimental import pallas as pl
from jax.experimental.pallas import tpu as pltpu
from jax.experimental.pallas import tpu_sc as plsc

D_MODEL = 128
_SCALE = math.sqrt(float(D_MODEL))

_NC = 2
_NS = 16
_NW = _NC * _NS

_NROWS = 4096
_K = 50
_CPW = _NROWS // _NW
_NBUF = 5

_mesh = plsc.VectorSubcoreMesh(core_axis_name="c", subcore_axis_name="s")


@pl.kernel(
    out_type=jax.ShapeDtypeStruct((_K, _NROWS, D_MODEL), jnp.float32),
    mesh=_mesh,
    scratch_types=(
        [pltpu.VMEM((_K, _CPW), jnp.int32)]
        + [pltpu.VMEM((_CPW, D_MODEL), jnp.float32) for _ in range(_NBUF)]
        + [pltpu.SemaphoreType.DMA for _ in range(2 * _NBUF)]
    ),
    compiler_params=pltpu.CompilerParams(use_tc_tiling_on_sc=True),
)
def _gather_scale(idx_hbm, table_hbm, out_hbm, idx_v, r0, r1, r2, r3, r4,
                  g0, g1, g2, g3, g4, s0, s1, s2, s3, s4):
    bufs = (r0, r1, r2, r3, r4)
    gsems = (g0, g1, g2, g3, g4)
    ssems = (s0, s1, s2, s3, s4)
    wid = lax.axis_index("s") * _NC + lax.axis_index("c")
    base_col = wid * _CPW
    scale = jnp.float32(_SCALE)

    pltpu.sync_copy(
        idx_hbm.at[pl.ds(0, _K), pl.ds(base_col, _CPW)], idx_v)

    def start_gather(k, b):
        pltpu.async_copy(table_hbm.at[idx_v.at[k]], bufs[b], gsems[b])

    def wait_gather(b):
        pltpu.make_async_copy(
            table_hbm.at[idx_v.at[0]], bufs[b], gsems[b]).wait()

    def wait_store(b):
        pltpu.make_async_copy(
            bufs[b], out_hbm.at[0, pl.ds(base_col, _CPW)], ssems[b]).wait()

    for c in range(_NBUF - 1):
        start_gather(c, c)

    @pl.loop(0, _K, step=_NBUF)
    def _chunks(j):
        for b in range(_NBUF):
            k = j + b
            nb = (b + _NBUF - 1) % _NBUF
            @pl.when(k + _NBUF - 1 < _K)
            def _():
                @pl.when(k >= 1)
                def _():
                    wait_store(nb)
                start_gather(k + _NBUF - 1, nb)

            wait_gather(b)

            @plsc.parallel_loop(0, _CPW, unroll=2)
            def _rows(i):
                for q in range(D_MODEL // 16):
                    sl = pl.ds(q * 16, 16)
                    bufs[b][i, sl] = bufs[b][i, sl] * scale

            pltpu.async_copy(
                bufs[b],
                out_hbm.at[k, pl.ds(base_col, _CPW)],
                ssems[b])

    for c in range(_K - _NBUF, _K):
        wait_store(c % _NBUF)


def kernel(idxs, emb_table):
    out_t = _gather_scale(idxs.T.astype(jnp.int32), emb_table)
    return out_t.transpose(1, 0, 2)
